# baseline (speedup 1.0000x reference)
import functools

import jax
import jax.numpy as jnp
from jax import lax
from jax.experimental import pallas as pl
from jax.experimental.pallas import tpu as pltpu
from jax.experimental.pallas import tpu_sc as plsc

_NUM_CORES = 2
_NUM_SUBCORES = 16
_NUM_WORKERS = _NUM_CORES * _NUM_SUBCORES


def kernel(x, position_embeddings):
    seq_len = x.shape[1]
    emb_dim = position_embeddings.shape[1]
    rows_per_w = seq_len // _NUM_WORKERS

    @functools.partial(
        pl.kernel,
        out_type=jax.ShapeDtypeStruct((seq_len, emb_dim), position_embeddings.dtype),
        mesh=plsc.VectorSubcoreMesh(core_axis_name="c", subcore_axis_name="s"),
    )
    def copy_rows(table_hbm, out_hbm):
        wid = lax.axis_index("s") * _NUM_CORES + lax.axis_index("c")
        base = wid * rows_per_w
        pltpu.sync_copy(
            table_hbm.at[pl.ds(base, rows_per_w)],
            out_hbm.at[pl.ds(base, rows_per_w)],
        )

    return copy_rows(position_embeddings)[None]


# speedup vs baseline: 24.1416x; 24.1416x over previous
import functools

import jax
import jax.numpy as jnp
from jax import lax
from jax.experimental import pallas as pl
from jax.experimental.pallas import tpu as pltpu
from jax.experimental.pallas import tpu_sc as plsc

_NUM_CORES = 2
_NUM_SUBCORES = 16
_NUM_WORKERS = _NUM_CORES * _NUM_SUBCORES


_CHUNK = 16


def kernel(x, position_embeddings):
    seq_len = x.shape[1]
    emb_dim = position_embeddings.shape[1]
    rows_per_w = seq_len // _NUM_WORKERS
    n_chunks = rows_per_w // _CHUNK

    @functools.partial(
        pl.kernel,
        out_type=jax.ShapeDtypeStruct((seq_len, emb_dim), position_embeddings.dtype),
        mesh=plsc.VectorSubcoreMesh(core_axis_name="c", subcore_axis_name="s"),
        scratch_types=[
            pltpu.VMEM((_CHUNK, emb_dim), jnp.float32),
            pltpu.VMEM((_CHUNK, emb_dim), jnp.float32),
            pltpu.SemaphoreType.DMA,
            pltpu.SemaphoreType.DMA,
            pltpu.SemaphoreType.DMA,
            pltpu.SemaphoreType.DMA,
        ],
    )
    def copy_rows(table_hbm, out_hbm, buf0, buf1, rs0, rs1, ws0, ws1):
        wid = lax.axis_index("s") * _NUM_CORES + lax.axis_index("c")
        base = wid * rows_per_w
        bufs = (buf0, buf1)
        rsems = (rs0, rs1)
        wsems = (ws0, ws1)

        def rd(i, b):
            return pltpu.make_async_copy(
                table_hbm.at[pl.ds(base + i * _CHUNK, _CHUNK)], bufs[b], rsems[b])

        def wr(i, b):
            return pltpu.make_async_copy(
                bufs[b], out_hbm.at[pl.ds(base + i * _CHUNK, _CHUNK)], wsems[b])

        rd(0, 0).start()
        for i in range(n_chunks):
            b = i % 2
            if i + 1 < n_chunks:
                b2 = (i + 1) % 2
                if i >= 1:
                    wr(i - 1, b2).wait()
                rd(i + 1, b2).start()
            rd(i, b).wait()
            wr(i, b).start()
        wr(n_chunks - 2, (n_chunks - 2) % 2).wait()
        wr(n_chunks - 1, (n_chunks - 1) % 2).wait()

    return copy_rows(position_embeddings)[None]
